# Initial kernel scaffold; baseline (speedup 1.0000x reference)
import functools

import jax
import jax.numpy as jnp
from jax import lax
from jax.experimental import pallas as pl
from jax.experimental.pallas import tpu as pltpu
from jax.experimental.pallas import tpu_sc as plsc

N, E, D, H = 10000, 320000, 128, 64
NC, NS, L = 2, 16, 16
NW = NC * NS
EPT = E // NW
CH = 80
NCH = EPT // CH
NG = CH // L
RPT = N // NS
WREL = 16

_mesh = plsc.VectorSubcoreMesh(core_axis_name="c", subcore_axis_name="s")
_f32 = jnp.float32


def _silu(v):
    return v * jax.nn.sigmoid(v)


def _proj_body(h_ref, wa_ref, wb_ref, a_ref, b_ref):
    hh = h_ref[...]
    a_ref[...] = jnp.dot(hh, wa_ref[...], preferred_element_type=_f32)
    b_ref[...] = jnp.dot(hh, wb_ref[...], preferred_element_type=_f32)


def _proj(h, wa, wb):
    blk = 500
    return pl.pallas_call(
        _proj_body,
        grid=(N // blk,),
        in_specs=[
            pl.BlockSpec((blk, D), lambda i: (i, 0)),
            pl.BlockSpec((D, H), lambda i: (0, 0)),
            pl.BlockSpec((D, H), lambda i: (0, 0)),
        ],
        out_specs=[
            pl.BlockSpec((blk, H), lambda i: (i, 0)),
            pl.BlockSpec((blk, H), lambda i: (i, 0)),
        ],
        out_shape=[
            jax.ShapeDtypeStruct((N, H), _f32),
            jax.ShapeDtypeStruct((N, H), _f32),
        ],
    )(h, wa, wb)


@functools.partial(
    pl.kernel,
    out_type=(
        jax.ShapeDtypeStruct((E, H), _f32),
        jax.ShapeDtypeStruct((E, H), _f32),
        jax.ShapeDtypeStruct((E * WREL,), _f32),
    ),
    mesh=_mesh,
    scratch_types=[
        pltpu.VMEM((3 * N,), _f32),
        pltpu.VMEM((CH,), jnp.int32),
        pltpu.VMEM((CH,), jnp.int32),
        pltpu.VMEM((CH, H), _f32),
        pltpu.VMEM((CH, H), _f32),
        pltpu.VMEM((CH * WREL,), _f32),
        pltpu.SemaphoreType.DMA,
    ],
)
def _sc_gather(a_hbm, b_hbm, xt_hbm, row_hbm, col_hbm,
               ar_hbm, bc_hbm, rw_hbm,
               xv, ir_v, ic_v, av, bv, rv, sem):
    wid = lax.axis_index("c") * NS + lax.axis_index("s")
    pltpu.sync_copy(xt_hbm, xv)

    def chunk(k, carry):
        base = wid * EPT + k * CH
        pltpu.sync_copy(row_hbm.at[pl.ds(base, CH)], ir_v)
        pltpu.sync_copy(col_hbm.at[pl.ds(base, CH)], ic_v)
        cp_a = pltpu.async_copy(a_hbm.at[ir_v], av, sem)
        cp_b = pltpu.async_copy(b_hbm.at[ic_v], bv, sem)

        def grp(g, c2):
            ir = ir_v[pl.ds(g * L, L)]
            ic = ic_v[pl.ds(g * L, L)]
            r0 = plsc.load_gather(xv, [ir]) - plsc.load_gather(xv, [ic])
            r1 = plsc.load_gather(xv, [ir + N]) - plsc.load_gather(xv, [ic + N])
            r2 = plsc.load_gather(xv, [ir + 2 * N]) - plsc.load_gather(xv, [ic + 2 * N])
            d2 = r0 * r0 + r1 * r1 + r2 * r2
            pos = (g * L + lax.iota(jnp.int32, L)) * WREL
            plsc.store_scatter(rv, [pos], r0)
            plsc.store_scatter(rv, [pos + 1], r1)
            plsc.store_scatter(rv, [pos + 2], r2)
            plsc.store_scatter(rv, [pos + 3], d2)
            return c2

        lax.fori_loop(0, NG, grp, 0)
        cp_a.wait()
        cp_b.wait()
        pltpu.sync_copy(av, ar_hbm.at[pl.ds(base, CH)])
        pltpu.sync_copy(bv, bc_hbm.at[pl.ds(base, CH)])
        pltpu.sync_copy(rv, rw_hbm.at[pl.ds(base * WREL, CH * WREL)])
        return carry

    lax.fori_loop(0, NCH, chunk, 0)


def _edge_body(ar_ref, bc_ref, rw_ref, be1_ref, we2_ref, be2_ref,
               wx1_ref, bx1_ref, wx2_ref, bx2_ref, wd2_ref,
               m_ref, wrel_ref):
    rw = rw_ref[...]
    s = ar_ref[...] + bc_ref[...] + be1_ref[...] + rw[:, 3:4] * wd2_ref[...]
    m = _silu(s)
    m = _silu(jnp.dot(m, we2_ref[...], preferred_element_type=_f32) + be2_ref[...])
    t = _silu(jnp.dot(m, wx1_ref[...], preferred_element_type=_f32) + bx1_ref[...])
    w_ij = jnp.dot(t, wx2_ref[...], preferred_element_type=_f32) + bx2_ref[...]
    m_ref[...] = m
    lane = lax.broadcasted_iota(jnp.int32, (1, WREL), 1)
    wrel_ref[...] = jnp.where(lane < 3, rw * w_ij, 0.0)


def _edge_mlp(ar, bc, rw, b_e1, W_e2, b_e2, W_x1, b_x1, W_x2, b_x2, w_d2):
    blk = 1000
    full = lambda shape: pl.BlockSpec(shape, lambda i: (0, 0))
    return pl.pallas_call(
        _edge_body,
        grid=(E // blk,),
        in_specs=[
            pl.BlockSpec((blk, H), lambda i: (i, 0)),
            pl.BlockSpec((blk, H), lambda i: (i, 0)),
            pl.BlockSpec((blk, WREL), lambda i: (i, 0)),
            full((1, H)), full((H, H)), full((1, H)),
            full((H, H)), full((1, H)), full((H, 1)), full((1, 1)),
            full((1, H)),
        ],
        out_specs=[
            pl.BlockSpec((blk, H), lambda i: (i, 0)),
            pl.BlockSpec((blk, WREL), lambda i: (i, 0)),
        ],
        out_shape=[
            jax.ShapeDtypeStruct((E, H), _f32),
            jax.ShapeDtypeStruct((E, WREL), _f32),
        ],
    )(ar, bc, rw, b_e1, W_e2, b_e2, W_x1, b_x1, W_x2, b_x2, w_d2)


@functools.partial(
    pl.kernel,
    out_type=(
        jax.ShapeDtypeStruct((NC, N, H), _f32),
        jax.ShapeDtypeStruct((NC, N, WREL), _f32),
    ),
    mesh=_mesh,
    scratch_types=[
        pltpu.VMEM_SHARED((N, H), _f32),
        pltpu.VMEM_SHARED((N, WREL), _f32),
        pltpu.VMEM((CH,), jnp.int32),
        pltpu.VMEM((CH, H), _f32),
        pltpu.VMEM((CH, WREL), _f32),
    ],
)
def _sc_scatter(m_hbm, wrel_hbm, row_hbm, z64_hbm, z16_hbm,
                agg_hbm, cdp_hbm,
                sh_m, sh_w, iv, mv, wv):
    cc = lax.axis_index("c")
    ss = lax.axis_index("s")
    wid = cc * NS + ss
    pltpu.sync_copy(z64_hbm.at[pl.ds(ss * RPT, RPT)], sh_m.at[pl.ds(ss * RPT, RPT)])
    pltpu.sync_copy(z16_hbm.at[pl.ds(ss * RPT, RPT)], sh_w.at[pl.ds(ss * RPT, RPT)])
    plsc.subcore_barrier()

    def chunk(k, carry):
        base = wid * EPT + k * CH
        pltpu.sync_copy(row_hbm.at[pl.ds(base, CH)], iv)
        pltpu.sync_copy(m_hbm.at[pl.ds(base, CH)], mv)
        pltpu.sync_copy(wrel_hbm.at[pl.ds(base, CH)], wv)
        pltpu.sync_copy(mv, sh_m.at[iv], add=True)
        pltpu.sync_copy(wv, sh_w.at[iv], add=True)
        return carry

    lax.fori_loop(0, NCH, chunk, 0)
    plsc.subcore_barrier()
    pltpu.sync_copy(sh_m.at[pl.ds(ss * RPT, RPT)],
                    agg_hbm.at[cc, pl.ds(ss * RPT, RPT)])
    pltpu.sync_copy(sh_w.at[pl.ds(ss * RPT, RPT)],
                    cdp_hbm.at[cc, pl.ds(ss * RPT, RPT)])


def _node_body(h_ref, a0_ref, a1_ref, c0_ref, c1_ref, x_ref, dinv_ref,
               wh1a_ref, wh1b_ref, bh1_ref, wh2_ref, bh2_ref, g_ref, b_ref,
               hn_ref, xn_ref):
    hh = h_ref[...]
    agg = a0_ref[...] + a1_ref[...]
    t = _silu(jnp.dot(hh, wh1a_ref[...], preferred_element_type=_f32)
              + jnp.dot(agg, wh1b_ref[...], preferred_element_type=_f32)
              + bh1_ref[...])
    hu = jnp.dot(t, wh2_ref[...], preferred_element_type=_f32) + bh2_ref[...]
    y = hh + hu
    mu = jnp.mean(y, axis=-1, keepdims=True)
    var = jnp.mean((y - mu) ** 2, axis=-1, keepdims=True)
    hn_ref[...] = (y - mu) * lax.rsqrt(var + 1e-5) * g_ref[...] + b_ref[...]
    cd = (c0_ref[...] + c1_ref[...])[:, :3]
    xn_ref[...] = x_ref[...] + cd * dinv_ref[...]


def _node(h, a0, a1, c0, c1, x, dinv, wh1a, wh1b, b_h1, W_h2, b_h2, ln_g, ln_b):
    blk = 500
    full = lambda shape: pl.BlockSpec(shape, lambda i: (0, 0))
    return pl.pallas_call(
        _node_body,
        grid=(N // blk,),
        in_specs=[
            pl.BlockSpec((blk, D), lambda i: (i, 0)),
            pl.BlockSpec((blk, H), lambda i: (i, 0)),
            pl.BlockSpec((blk, H), lambda i: (i, 0)),
            pl.BlockSpec((blk, WREL), lambda i: (i, 0)),
            pl.BlockSpec((blk, WREL), lambda i: (i, 0)),
            pl.BlockSpec((blk, 3), lambda i: (i, 0)),
            pl.BlockSpec((blk, 1), lambda i: (i, 0)),
            full((D, H)), full((H, H)), full((1, H)),
            full((H, D)), full((1, D)), full((1, D)), full((1, D)),
        ],
        out_specs=[
            pl.BlockSpec((blk, D), lambda i: (i, 0)),
            pl.BlockSpec((blk, 3), lambda i: (i, 0)),
        ],
        out_shape=[
            jax.ShapeDtypeStruct((N, D), _f32),
            jax.ShapeDtypeStruct((N, 3), _f32),
        ],
    )(h, a0, a1, c0, c1, x, dinv, wh1a, wh1b, b_h1, W_h2, b_h2, ln_g, ln_b)


def kernel(h, x, edge_index, degree_inv,
           W_e1, b_e1, W_e2, b_e2, W_h1, b_h1, W_h2, b_h2,
           W_x1, b_x1, W_x2, b_x2, ln_g, ln_b):
    row = edge_index[0].astype(jnp.int32)
    col = edge_index[1].astype(jnp.int32)
    xt = x.T.reshape(-1)

    A, B = _proj(h, W_e1[:D], W_e1[D:2 * D])
    ar, bc, rw_flat = _sc_gather(A, B, xt, row, col)
    rw = rw_flat.reshape(E, WREL)
    m, wrel = _edge_mlp(ar, bc, rw,
                        b_e1.reshape(1, H), W_e2, b_e2.reshape(1, H),
                        W_x1, b_x1.reshape(1, H), W_x2, b_x2.reshape(1, 1),
                        W_e1[2 * D].reshape(1, H))
    z64 = jnp.zeros((N, H), _f32)
    z16 = jnp.zeros((N, WREL), _f32)
    aggp, cdp = _sc_scatter(m, wrel, row, z64, z16)
    h_new, x_new = _node(h, aggp[0], aggp[1], cdp[0], cdp[1], x,
                         degree_inv.reshape(N, 1),
                         W_h1[:D], W_h1[D:], b_h1.reshape(1, H),
                         W_h2, b_h2.reshape(1, D), ln_g.reshape(1, D),
                         ln_b.reshape(1, D))
    return (h_new, x_new)


# speedup vs baseline: 4.6041x; 4.6041x over previous
import functools

import jax
import jax.numpy as jnp
from jax import lax
from jax.experimental import pallas as pl
from jax.experimental.pallas import tpu as pltpu
from jax.experimental.pallas import tpu_sc as plsc

N, E, D, H = 10000, 320000, 128, 64
NC, NS, L = 2, 16, 16
NW = NC * NS
EPT = E // NW
CH = 80
NCH = EPT // CH
W = 128
NP = 10240
RPT = NP // NS

_mesh = plsc.VectorSubcoreMesh(core_axis_name="c", subcore_axis_name="s")
_f32 = jnp.float32


def _silu(v):
    return v * jax.nn.sigmoid(v)


def _proj_body(h_ref, x_ref, wa_ref, wb_ref, t1_ref, t2_ref):
    hh = h_ref[...]
    xb = x_ref[...]
    pad = jnp.zeros((hh.shape[0], W - H - 3), _f32)
    a = jnp.dot(hh, wa_ref[...], preferred_element_type=_f32)
    b = jnp.dot(hh, wb_ref[...], preferred_element_type=_f32)
    t1_ref[...] = jnp.concatenate([a, xb, pad], axis=1)
    t2_ref[...] = jnp.concatenate([b, xb, pad], axis=1)


def _proj(h, x, wa, wb):
    blk = 400
    return pl.pallas_call(
        _proj_body,
        grid=(N // blk,),
        in_specs=[
            pl.BlockSpec((blk, D), lambda i: (i, 0)),
            pl.BlockSpec((blk, 3), lambda i: (i, 0)),
            pl.BlockSpec((D, H), lambda i: (0, 0)),
            pl.BlockSpec((D, H), lambda i: (0, 0)),
        ],
        out_specs=[
            pl.BlockSpec((blk, W), lambda i: (i, 0)),
            pl.BlockSpec((blk, W), lambda i: (i, 0)),
        ],
        out_shape=[
            jax.ShapeDtypeStruct((N, W), _f32),
            jax.ShapeDtypeStruct((N, W), _f32),
        ],
    )(h, x, wa, wb)


@functools.partial(
    pl.kernel,
    out_type=jax.ShapeDtypeStruct((E, W), _f32),
    mesh=_mesh,
    scratch_types=[
        pltpu.VMEM((CH,), jnp.int32),
        pltpu.VMEM((CH,), jnp.int32),
        pltpu.VMEM((CH, W), _f32),
        pltpu.VMEM((CH, W), _f32),
        pltpu.VMEM((CH, W), _f32),
        pltpu.SemaphoreType.DMA,
        pltpu.SemaphoreType.DMA,
    ],
)
def _sc_gather(t1_hbm, t2_hbm, row_hbm, col_hbm, p_hbm,
               ir_v, ic_v, av, bv, pv, sem, sem_wb):
    wid = lax.axis_index("c") * NS + lax.axis_index("s")
    zero = jnp.zeros((L,), _f32)

    def zrow(i, carry):
        pv[i, pl.ds(5 * L, L)] = zero
        pv[i, pl.ds(6 * L, L)] = zero
        pv[i, pl.ds(7 * L, L)] = zero
        return carry

    lax.fori_loop(0, CH, zrow, 0)

    def chunk(k, carry):
        base = wid * EPT + k * CH
        ci = pltpu.async_copy(row_hbm.at[pl.ds(base, CH)], ir_v, sem)
        cj = pltpu.async_copy(col_hbm.at[pl.ds(base, CH)], ic_v, sem)
        ci.wait()
        cj.wait()
        cp_a = pltpu.async_copy(t1_hbm.at[ir_v], av, sem)
        cp_b = pltpu.async_copy(t2_hbm.at[ic_v], bv, sem)
        cp_a.wait()
        cp_b.wait()

        @pl.when(k > 0)
        def _drain():
            pltpu.make_async_copy(p_hbm.at[pl.ds(0, CH)], pv, sem_wb).wait()

        def pack_row(i, c2):
            for j in range(H // L):
                sl = pl.ds(j * L, L)
                pv[i, sl] = av[i, sl] + bv[i, sl]
            sl = pl.ds(H, L)
            pv[i, sl] = av[i, sl] - bv[i, sl]
            return c2

        lax.fori_loop(0, CH, pack_row, 0)
        pltpu.async_copy(pv, p_hbm.at[pl.ds(base, CH)], sem_wb)
        return carry

    lax.fori_loop(0, NCH, chunk, 0)
    pltpu.make_async_copy(p_hbm.at[pl.ds(0, CH)], pv, sem_wb).wait()


def _edge_body(p_ref, be1_ref, we2_ref, be2_ref,
               wx1_ref, bx1_ref, wx2_ref, bx2_ref, wd2_ref, m_ref):
    p = p_ref[...]
    rel = p[:, H:H + 16]
    d2 = jnp.sum(rel * rel, axis=1, keepdims=True)
    s = p[:, :H] + be1_ref[...] + d2 * wd2_ref[...]
    m = _silu(s)
    m = _silu(jnp.dot(m, we2_ref[...], preferred_element_type=_f32) + be2_ref[...])
    t = _silu(jnp.dot(m, wx1_ref[...], preferred_element_type=_f32) + bx1_ref[...])
    w_ij = jnp.dot(t, wx2_ref[...], preferred_element_type=_f32) + bx2_ref[...]
    pad = jnp.zeros((p.shape[0], W - H - 16), _f32)
    m_ref[...] = jnp.concatenate([m, rel * w_ij, pad], axis=1)


def _edge_mlp(p, b_e1, W_e2, b_e2, W_x1, b_x1, W_x2, b_x2, w_d2):
    blk = 1000
    full = lambda shape: pl.BlockSpec(shape, lambda i: (0, 0))
    return pl.pallas_call(
        _edge_body,
        grid=(E // blk,),
        in_specs=[
            pl.BlockSpec((blk, W), lambda i: (i, 0)),
            full((1, H)), full((H, H)), full((1, H)),
            full((H, H)), full((1, H)), full((H, 1)), full((1, 1)),
            full((1, H)),
        ],
        out_specs=pl.BlockSpec((blk, W), lambda i: (i, 0)),
        out_shape=jax.ShapeDtypeStruct((E, W), _f32),
    )(p, b_e1, W_e2, b_e2, W_x1, b_x1, W_x2, b_x2, w_d2)


@functools.partial(
    pl.kernel,
    out_type=jax.ShapeDtypeStruct((NC, NP, W), _f32),
    mesh=_mesh,
    scratch_types=[
        pltpu.VMEM_SHARED((NP, W), _f32),
        pltpu.VMEM((CH,), jnp.int32),
        pltpu.VMEM((CH, W), _f32),
    ],
)
def _sc_scatter(m_hbm, row_hbm, z_hbm, agg_hbm, sh, iv, mv):
    cc = lax.axis_index("c")
    ss = lax.axis_index("s")
    wid = cc * NS + ss
    pltpu.sync_copy(z_hbm.at[pl.ds(ss * RPT, RPT)], sh.at[pl.ds(ss * RPT, RPT)])
    plsc.subcore_barrier()

    def chunk(k, carry):
        base = wid * EPT + k * CH
        pltpu.sync_copy(row_hbm.at[pl.ds(base, CH)], iv)
        pltpu.sync_copy(m_hbm.at[pl.ds(base, CH)], mv)
        pltpu.sync_copy(mv, sh.at[iv], add=True)
        return carry

    lax.fori_loop(0, NCH, chunk, 0)
    plsc.subcore_barrier()
    pltpu.sync_copy(sh.at[pl.ds(ss * RPT, RPT)],
                    agg_hbm.at[cc, pl.ds(ss * RPT, RPT)])


def _node_body(h_ref, a0_ref, a1_ref, x_ref, dinv_ref,
               wh1a_ref, wh1b_ref, bh1_ref, wh2_ref, bh2_ref, g_ref, b_ref,
               hn_ref, xn_ref):
    hh = h_ref[...]
    acc = a0_ref[...] + a1_ref[...]
    agg = acc[:, :H]
    t = _silu(jnp.dot(hh, wh1a_ref[...], preferred_element_type=_f32)
              + jnp.dot(agg, wh1b_ref[...], preferred_element_type=_f32)
              + bh1_ref[...])
    hu = jnp.dot(t, wh2_ref[...], preferred_element_type=_f32) + bh2_ref[...]
    y = hh + hu
    mu = jnp.mean(y, axis=-1, keepdims=True)
    var = jnp.mean((y - mu) ** 2, axis=-1, keepdims=True)
    hn_ref[...] = (y - mu) * lax.rsqrt(var + 1e-5) * g_ref[...] + b_ref[...]
    xn_ref[...] = x_ref[...] + acc[:, H:H + 3] * dinv_ref[...]


def _node(h, a0, a1, x, dinv, wh1a, wh1b, b_h1, W_h2, b_h2, ln_g, ln_b):
    blk = 400
    full = lambda shape: pl.BlockSpec(shape, lambda i: (0, 0))
    return pl.pallas_call(
        _node_body,
        grid=(N // blk,),
        in_specs=[
            pl.BlockSpec((blk, D), lambda i: (i, 0)),
            pl.BlockSpec((blk, W), lambda i: (i, 0)),
            pl.BlockSpec((blk, W), lambda i: (i, 0)),
            pl.BlockSpec((blk, 3), lambda i: (i, 0)),
            pl.BlockSpec((blk, 1), lambda i: (i, 0)),
            full((D, H)), full((H, H)), full((1, H)),
            full((H, D)), full((1, D)), full((1, D)), full((1, D)),
        ],
        out_specs=[
            pl.BlockSpec((blk, D), lambda i: (i, 0)),
            pl.BlockSpec((blk, 3), lambda i: (i, 0)),
        ],
        out_shape=[
            jax.ShapeDtypeStruct((N, D), _f32),
            jax.ShapeDtypeStruct((N, 3), _f32),
        ],
    )(h, a0, a1, x, dinv, wh1a, wh1b, b_h1, W_h2, b_h2, ln_g, ln_b)


def kernel(h, x, edge_index, degree_inv,
           W_e1, b_e1, W_e2, b_e2, W_h1, b_h1, W_h2, b_h2,
           W_x1, b_x1, W_x2, b_x2, ln_g, ln_b):
    row = edge_index[0].astype(jnp.int32)
    col = edge_index[1].astype(jnp.int32)

    t1, t2 = _proj(h, x, W_e1[:D], W_e1[D:2 * D])
    p = _sc_gather(t1, t2, row, col)
    m = _edge_mlp(p, b_e1.reshape(1, H), W_e2, b_e2.reshape(1, H),
                  W_x1, b_x1.reshape(1, H), W_x2, b_x2.reshape(1, 1),
                  W_e1[2 * D].reshape(1, H))
    z = jnp.zeros((NP, W), _f32)
    aggp = _sc_scatter(m, row, z)
    h_new, x_new = _node(h, aggp[0, :N], aggp[1, :N], x,
                         degree_inv.reshape(N, 1),
                         W_h1[:D], W_h1[D:], b_h1.reshape(1, H),
                         W_h2, b_h2.reshape(1, D), ln_g.reshape(1, D),
                         ln_b.reshape(1, D))
    return (h_new, x_new)


# speedup vs baseline: 7.7508x; 1.6835x over previous
import functools

import jax
import jax.numpy as jnp
from jax import lax
from jax.experimental import pallas as pl
from jax.experimental.pallas import tpu as pltpu
from jax.experimental.pallas import tpu_sc as plsc

N, E, D, H = 10000, 320000, 128, 64
NC, NS, L = 2, 16, 16
NW = NC * NS
EPT = E // NW
CH = 80
NCH = EPT // CH
W = 128
NP = 10240
RPT = NP // NS

_mesh = plsc.VectorSubcoreMesh(core_axis_name="c", subcore_axis_name="s")
_f32 = jnp.float32


def _silu(v):
    return v / (1.0 + jnp.exp(-v))


def _proj_body(h_ref, x_ref, wa_ref, wb_ref, t1_ref, t2_ref):
    hh = h_ref[...]
    xb = x_ref[...]
    pad = jnp.zeros((hh.shape[0], W - H - 3), _f32)
    a = jnp.dot(hh, wa_ref[...], preferred_element_type=_f32)
    b = jnp.dot(hh, wb_ref[...], preferred_element_type=_f32)
    t1_ref[...] = jnp.concatenate([a, xb, pad], axis=1)
    t2_ref[...] = jnp.concatenate([b, xb, pad], axis=1)


def _proj(h, x, wa, wb):
    blk = 400
    return pl.pallas_call(
        _proj_body,
        grid=(N // blk,),
        in_specs=[
            pl.BlockSpec((blk, D), lambda i: (i, 0)),
            pl.BlockSpec((blk, 3), lambda i: (i, 0)),
            pl.BlockSpec((D, H), lambda i: (0, 0)),
            pl.BlockSpec((D, H), lambda i: (0, 0)),
        ],
        out_specs=[
            pl.BlockSpec((blk, W), lambda i: (i, 0)),
            pl.BlockSpec((blk, W), lambda i: (i, 0)),
        ],
        out_shape=[
            jax.ShapeDtypeStruct((N, W), _f32),
            jax.ShapeDtypeStruct((N, W), _f32),
        ],
    )(h, x, wa, wb)


@functools.partial(
    pl.kernel,
    out_type=jax.ShapeDtypeStruct((E, W), _f32),
    mesh=_mesh,
    scratch_types=[
        pltpu.VMEM((EPT,), jnp.int32),
        pltpu.VMEM((EPT,), jnp.int32),
        pltpu.VMEM((2, CH, W), _f32),
        pltpu.VMEM((2, CH, W), _f32),
        pltpu.VMEM((2, CH, W), _f32),
        pltpu.SemaphoreType.DMA,
        pltpu.SemaphoreType.DMA,
        pltpu.SemaphoreType.DMA,
        pltpu.SemaphoreType.DMA,
    ],
)
def _sc_gather(t1_hbm, t2_hbm, row_hbm, col_hbm, p_hbm,
               irf, icf, avv, bvv, pvv, sg0, sg1, sw0, sw1):
    wid = lax.axis_index("c") * NS + lax.axis_index("s")
    zero = jnp.zeros((L,), _f32)
    sgs = (sg0, sg1)
    sws = (sw0, sw1)

    def zrow(i, carry):
        for b in range(2):
            pvv[b, i, pl.ds(5 * L, L)] = zero
            pvv[b, i, pl.ds(6 * L, L)] = zero
            pvv[b, i, pl.ds(7 * L, L)] = zero
        return carry

    lax.fori_loop(0, CH, zrow, 0)
    pltpu.sync_copy(row_hbm.at[pl.ds(wid * EPT, EPT)], irf)
    pltpu.sync_copy(col_hbm.at[pl.ds(wid * EPT, EPT)], icf)

    def issue_gather(k, b):
        pltpu.async_copy(t1_hbm.at[irf.at[pl.ds(k * CH, CH)]], avv.at[b], sgs[b])
        pltpu.async_copy(t2_hbm.at[icf.at[pl.ds(k * CH, CH)]], bvv.at[b], sgs[b])

    def wait_gather(b):
        pltpu.make_async_copy(t1_hbm.at[pl.ds(0, CH)], avv.at[b], sgs[b]).wait()
        pltpu.make_async_copy(t2_hbm.at[pl.ds(0, CH)], bvv.at[b], sgs[b]).wait()

    def drain_wb(b):
        pltpu.make_async_copy(p_hbm.at[pl.ds(0, CH)], pvv.at[b], sws[b]).wait()

    def pack(b):
        av = avv.at[b]
        bv = bvv.at[b]
        pv = pvv.at[b]

        def pack_row(i, c2):
            for j in range(H // L):
                sl = pl.ds(j * L, L)
                pv[i, sl] = av[i, sl] + bv[i, sl]
            sl = pl.ds(H, L)
            pv[i, sl] = av[i, sl] - bv[i, sl]
            return c2

        lax.fori_loop(0, CH, pack_row, 0)

    def process(k, b, drain):
        wait_gather(b)

        @pl.when(drain)
        def _():
            drain_wb(b)

        pack(b)
        pltpu.async_copy(pvv.at[b], p_hbm.at[pl.ds(wid * EPT + k * CH, CH)],
                         sws[b])

    issue_gather(0, 0)
    issue_gather(1, 1)

    def pair(p_idx, carry):
        k0 = 2 * p_idx
        process(k0, 0, p_idx > 0)
        issue_gather(k0 + 2, 0)

        process(k0 + 1, 1, p_idx > 0)

        @pl.when(k0 + 3 < NCH)
        def _():
            issue_gather(k0 + 3, 1)

        return carry

    lax.fori_loop(0, (NCH - 1) // 2, pair, 0)
    process(NCH - 1, 0, True)
    drain_wb(0)
    drain_wb(1)


def _edge_body(p_ref, be1_ref, we2_ref, be2_ref,
               wx1_ref, bx1_ref, wx2_ref, bx2_ref, wd2_ref, m_ref):
    p = p_ref[...]
    rel = p[:, H:H + 16]
    s = (p[:, :H] + be1_ref[...]
         + jnp.dot(rel * rel, wd2_ref[...], preferred_element_type=_f32))
    m = _silu(s)
    m = _silu(jnp.dot(m, we2_ref[...], preferred_element_type=_f32) + be2_ref[...])
    t = _silu(jnp.dot(m, wx1_ref[...], preferred_element_type=_f32) + bx1_ref[...])
    w_ij = jnp.dot(t, wx2_ref[...], preferred_element_type=_f32) + bx2_ref[...]
    pad = jnp.zeros((p.shape[0], W - H - 16), _f32)
    m_ref[...] = jnp.concatenate([m, rel * w_ij, pad], axis=1)


def _edge_mlp(p, b_e1, W_e2, b_e2, W_x1, b_x1, W_x2, b_x2, wd2_16):
    blk = 2000
    full = lambda shape: pl.BlockSpec(shape, lambda i: (0, 0))
    return pl.pallas_call(
        _edge_body,
        grid=(E // blk,),
        in_specs=[
            pl.BlockSpec((blk, W), lambda i: (i, 0)),
            full((1, H)), full((H, H)), full((1, H)),
            full((H, H)), full((1, H)), full((H, 1)), full((1, 1)),
            full((16, H)),
        ],
        out_specs=pl.BlockSpec((blk, W), lambda i: (i, 0)),
        out_shape=jax.ShapeDtypeStruct((E, W), _f32),
    )(p, b_e1, W_e2, b_e2, W_x1, b_x1, W_x2, b_x2, wd2_16)


@functools.partial(
    pl.kernel,
    out_type=jax.ShapeDtypeStruct((NC, NP, W), _f32),
    mesh=_mesh,
    scratch_types=[
        pltpu.VMEM_SHARED((NP, W), _f32),
        pltpu.VMEM((2, CH), jnp.int32),
        pltpu.VMEM((2, CH, W), _f32),
        pltpu.SemaphoreType.DMA,
        pltpu.SemaphoreType.DMA,
    ],
)
def _sc_scatter(m_hbm, row_hbm, z_hbm, agg_hbm, sh, ivv, mvv, sl0, sl1):
    cc = lax.axis_index("c")
    ss = lax.axis_index("s")
    wid = cc * NS + ss
    sls = (sl0, sl1)
    pltpu.sync_copy(z_hbm.at[pl.ds(ss * RPT, RPT)], sh.at[pl.ds(ss * RPT, RPT)])
    plsc.subcore_barrier()

    def issue_load(k, b):
        base = wid * EPT + k * CH
        pltpu.async_copy(row_hbm.at[pl.ds(base, CH)], ivv.at[b], sls[b])
        pltpu.async_copy(m_hbm.at[pl.ds(base, CH)], mvv.at[b], sls[b])

    def wait_load(b):
        pltpu.make_async_copy(row_hbm.at[pl.ds(0, CH)], ivv.at[b], sls[b]).wait()
        pltpu.make_async_copy(m_hbm.at[pl.ds(0, CH)], mvv.at[b], sls[b]).wait()

    def scat(b):
        pltpu.sync_copy(mvv.at[b], sh.at[ivv.at[b]], add=True)

    issue_load(0, 0)
    issue_load(1, 1)

    def pair(p_idx, carry):
        k0 = 2 * p_idx
        wait_load(0)
        scat(0)
        issue_load(k0 + 2, 0)
        wait_load(1)
        scat(1)

        @pl.when(k0 + 3 < NCH)
        def _():
            issue_load(k0 + 3, 1)

        return carry

    lax.fori_loop(0, (NCH - 1) // 2, pair, 0)
    wait_load(0)
    scat(0)
    plsc.subcore_barrier()
    pltpu.sync_copy(sh.at[pl.ds(ss * RPT, RPT)],
                    agg_hbm.at[cc, pl.ds(ss * RPT, RPT)])


def _node_body(h_ref, a0_ref, a1_ref, x_ref, dinv_ref,
               wh1a_ref, wh1b_ref, bh1_ref, wh2_ref, bh2_ref, g_ref, b_ref,
               hn_ref, xn_ref):
    hh = h_ref[...]
    acc = a0_ref[...] + a1_ref[...]
    agg = acc[:, :H]
    t = _silu(jnp.dot(hh, wh1a_ref[...], preferred_element_type=_f32)
              + jnp.dot(agg, wh1b_ref[...], preferred_element_type=_f32)
              + bh1_ref[...])
    hu = jnp.dot(t, wh2_ref[...], preferred_element_type=_f32) + bh2_ref[...]
    y = hh + hu
    mu = jnp.mean(y, axis=-1, keepdims=True)
    var = jnp.mean((y - mu) ** 2, axis=-1, keepdims=True)
    hn_ref[...] = (y - mu) * lax.rsqrt(var + 1e-5) * g_ref[...] + b_ref[...]
    xn_ref[...] = x_ref[...] + acc[:, H:H + 3] * dinv_ref[...]


def _node(h, a0, a1, x, dinv, wh1a, wh1b, b_h1, W_h2, b_h2, ln_g, ln_b):
    blk = 400
    full = lambda shape: pl.BlockSpec(shape, lambda i: (0, 0))
    return pl.pallas_call(
        _node_body,
        grid=(N // blk,),
        in_specs=[
            pl.BlockSpec((blk, D), lambda i: (i, 0)),
            pl.BlockSpec((blk, W), lambda i: (i, 0)),
            pl.BlockSpec((blk, W), lambda i: (i, 0)),
            pl.BlockSpec((blk, 3), lambda i: (i, 0)),
            pl.BlockSpec((blk, 1), lambda i: (i, 0)),
            full((D, H)), full((H, H)), full((1, H)),
            full((H, D)), full((1, D)), full((1, D)), full((1, D)),
        ],
        out_specs=[
            pl.BlockSpec((blk, D), lambda i: (i, 0)),
            pl.BlockSpec((blk, 3), lambda i: (i, 0)),
        ],
        out_shape=[
            jax.ShapeDtypeStruct((N, D), _f32),
            jax.ShapeDtypeStruct((N, 3), _f32),
        ],
    )(h, a0, a1, x, dinv, wh1a, wh1b, b_h1, W_h2, b_h2, ln_g, ln_b)


def kernel(h, x, edge_index, degree_inv,
           W_e1, b_e1, W_e2, b_e2, W_h1, b_h1, W_h2, b_h2,
           W_x1, b_x1, W_x2, b_x2, ln_g, ln_b):
    row = edge_index[0].astype(jnp.int32)
    col = edge_index[1].astype(jnp.int32)

    t1, t2 = _proj(h, x, W_e1[:D], W_e1[D:2 * D])
    p = _sc_gather(t1, t2, row, col)
    m = _edge_mlp(p, b_e1.reshape(1, H), W_e2, b_e2.reshape(1, H),
                  W_x1, b_x1.reshape(1, H), W_x2, b_x2.reshape(1, 1),
                  jnp.tile(W_e1[2 * D].reshape(1, H), (16, 1)))
    z = jnp.zeros((NP, W), _f32)
    aggp = _sc_scatter(m, row, z)
    h_new, x_new = _node(h, aggp[0, :N], aggp[1, :N], x,
                         degree_inv.reshape(N, 1),
                         W_h1[:D], W_h1[D:], b_h1.reshape(1, H),
                         W_h2, b_h2.reshape(1, D), ln_g.reshape(1, D),
                         ln_b.reshape(1, D))
    return (h_new, x_new)


# speedup vs baseline: 7.9012x; 1.0194x over previous
import functools

import jax
import jax.numpy as jnp
from jax import lax
from jax.experimental import pallas as pl
from jax.experimental.pallas import tpu as pltpu
from jax.experimental.pallas import tpu_sc as plsc

N, E, D, H = 10000, 320000, 128, 64
NC, NS, L = 2, 16, 16
NW = NC * NS
EPT = E // NW
CH = 80
NCH = EPT // CH
W = 128
NP = 10240
RPT = NP // NS

_mesh = plsc.VectorSubcoreMesh(core_axis_name="c", subcore_axis_name="s")
_f32 = jnp.float32


def _silu(v):
    return v / (1.0 + jnp.exp(-v))


def _proj_body(h_ref, x_ref, wa_ref, wb_ref, t1_ref, t2_ref):
    hh = h_ref[...]
    xb = x_ref[...]
    pad = jnp.zeros((hh.shape[0], W - H - 3), _f32)
    a = jnp.dot(hh, wa_ref[...], preferred_element_type=_f32)
    b = jnp.dot(hh, wb_ref[...], preferred_element_type=_f32)
    t1_ref[...] = jnp.concatenate([a, xb, pad], axis=1)
    t2_ref[...] = jnp.concatenate([b, xb, pad], axis=1)


def _proj(h, x, wa, wb):
    blk = 400
    return pl.pallas_call(
        _proj_body,
        grid=(N // blk,),
        in_specs=[
            pl.BlockSpec((blk, D), lambda i: (i, 0)),
            pl.BlockSpec((blk, 3), lambda i: (i, 0)),
            pl.BlockSpec((D, H), lambda i: (0, 0)),
            pl.BlockSpec((D, H), lambda i: (0, 0)),
        ],
        out_specs=[
            pl.BlockSpec((blk, W), lambda i: (i, 0)),
            pl.BlockSpec((blk, W), lambda i: (i, 0)),
        ],
        out_shape=[
            jax.ShapeDtypeStruct((N, W), _f32),
            jax.ShapeDtypeStruct((N, W), _f32),
        ],
    )(h, x, wa, wb)


@functools.partial(
    pl.kernel,
    out_type=jax.ShapeDtypeStruct((E, W), _f32),
    mesh=_mesh,
    scratch_types=[
        pltpu.VMEM((EPT,), jnp.int32),
        pltpu.VMEM((EPT,), jnp.int32),
        pltpu.VMEM((2, CH, W), _f32),
        pltpu.VMEM((2, CH, W), _f32),
        pltpu.VMEM((2, CH, W), _f32),
        pltpu.SemaphoreType.DMA,
        pltpu.SemaphoreType.DMA,
        pltpu.SemaphoreType.DMA,
        pltpu.SemaphoreType.DMA,
    ],
)
def _sc_gather(t1_hbm, t2_hbm, row_hbm, col_hbm, p_hbm,
               irf, icf, avv, bvv, pvv, sg0, sg1, sw0, sw1):
    wid = lax.axis_index("c") * NS + lax.axis_index("s")
    zero = jnp.zeros((L,), _f32)
    sgs = (sg0, sg1)
    sws = (sw0, sw1)

    def zrow(i, carry):
        for b in range(2):
            pvv[b, i, pl.ds(5 * L, L)] = zero
            pvv[b, i, pl.ds(6 * L, L)] = zero
            pvv[b, i, pl.ds(7 * L, L)] = zero
        return carry

    lax.fori_loop(0, CH, zrow, 0)
    pltpu.sync_copy(row_hbm.at[pl.ds(wid * EPT, EPT)], irf)
    pltpu.sync_copy(col_hbm.at[pl.ds(wid * EPT, EPT)], icf)

    def issue_gather(k, b):
        pltpu.async_copy(t1_hbm.at[irf.at[pl.ds(k * CH, CH)]], avv.at[b], sgs[b])
        pltpu.async_copy(t2_hbm.at[icf.at[pl.ds(k * CH, CH)]], bvv.at[b], sgs[b])

    def wait_gather(b):
        pltpu.make_async_copy(t1_hbm.at[pl.ds(0, CH)], avv.at[b], sgs[b]).wait()
        pltpu.make_async_copy(t2_hbm.at[pl.ds(0, CH)], bvv.at[b], sgs[b]).wait()

    def drain_wb(b):
        pltpu.make_async_copy(p_hbm.at[pl.ds(0, CH)], pvv.at[b], sws[b]).wait()

    def pack(b):
        av = avv.at[b]
        bv = bvv.at[b]
        pv = pvv.at[b]

        def pack_row(i, c2):
            for j in range(H // L):
                sl = pl.ds(j * L, L)
                pv[i, sl] = av[i, sl] + bv[i, sl]
            sl = pl.ds(H, L)
            pv[i, sl] = av[i, sl] - bv[i, sl]
            return c2

        lax.fori_loop(0, CH, pack_row, 0)

    def process(k, b, drain):
        wait_gather(b)

        @pl.when(drain)
        def _():
            drain_wb(b)

        pack(b)
        pltpu.async_copy(pvv.at[b], p_hbm.at[pl.ds(wid * EPT + k * CH, CH)],
                         sws[b])

    issue_gather(0, 0)
    issue_gather(1, 1)

    def pair(p_idx, carry):
        k0 = 2 * p_idx
        process(k0, 0, p_idx > 0)
        issue_gather(k0 + 2, 0)

        process(k0 + 1, 1, p_idx > 0)

        @pl.when(k0 + 3 < NCH)
        def _():
            issue_gather(k0 + 3, 1)

        return carry

    lax.fori_loop(0, (NCH - 1) // 2, pair, 0)
    process(NCH - 1, 0, True)
    drain_wb(0)
    drain_wb(1)


def _edge_body(p_ref, be1_ref, we2_ref, be2_ref,
               wx1_ref, bx1_ref, wx2_ref, bx2_ref, wd2_ref, m_ref):
    p = p_ref[...]
    rel = p[:, H:H + 16]
    s = (p[:, :H] + be1_ref[...]
         + jnp.dot(rel * rel, wd2_ref[...], preferred_element_type=_f32))
    m = _silu(s)
    m = _silu(jnp.dot(m, we2_ref[...], preferred_element_type=_f32) + be2_ref[...])
    t = _silu(jnp.dot(m, wx1_ref[...], preferred_element_type=_f32) + bx1_ref[...])
    w_ij = jnp.dot(t, wx2_ref[...], preferred_element_type=_f32) + bx2_ref[...]
    pad = jnp.zeros((p.shape[0], W - H - 16), _f32)
    m_ref[...] = jnp.concatenate([m, rel * w_ij, pad], axis=1)


def _edge_mlp(p, b_e1, W_e2, b_e2, W_x1, b_x1, W_x2, b_x2, wd2_16):
    blk = 2000
    full = lambda shape: pl.BlockSpec(shape, lambda i: (0, 0))
    return pl.pallas_call(
        _edge_body,
        grid=(E // blk,),
        in_specs=[
            pl.BlockSpec((blk, W), lambda i: (i, 0)),
            full((1, H)), full((H, H)), full((1, H)),
            full((H, H)), full((1, H)), full((H, 1)), full((1, 1)),
            full((16, H)),
        ],
        out_specs=pl.BlockSpec((blk, W), lambda i: (i, 0)),
        out_shape=jax.ShapeDtypeStruct((E, W), _f32),
    )(p, b_e1, W_e2, b_e2, W_x1, b_x1, W_x2, b_x2, wd2_16)


@functools.partial(
    pl.kernel,
    out_type=jax.ShapeDtypeStruct((NC, NP, W), _f32),
    mesh=_mesh,
    scratch_types=[
        pltpu.VMEM_SHARED((NP, W), _f32),
        pltpu.VMEM((2, CH), jnp.int32),
        pltpu.VMEM((2, CH, W), _f32),
        pltpu.SemaphoreType.DMA,
        pltpu.SemaphoreType.DMA,
    ],
)
def _sc_scatter(m_hbm, row_hbm, agg_hbm, sh, ivv, mvv, sl0, sl1):
    cc = lax.axis_index("c")
    ss = lax.axis_index("s")
    wid = cc * NS + ss
    sls = (sl0, sl1)
    zero = jnp.zeros((L,), _f32)

    def zrow(i, carry):
        for j in range(W // L):
            mvv[0, i, pl.ds(j * L, L)] = zero
        return carry

    lax.fori_loop(0, CH, zrow, 0)
    for r in range(RPT // CH):
        pltpu.sync_copy(mvv.at[0], sh.at[pl.ds(ss * RPT + r * CH, CH)])
    plsc.subcore_barrier()

    def issue_load(k, b):
        base = wid * EPT + k * CH
        pltpu.async_copy(row_hbm.at[pl.ds(base, CH)], ivv.at[b], sls[b])
        pltpu.async_copy(m_hbm.at[pl.ds(base, CH)], mvv.at[b], sls[b])

    def wait_load(b):
        pltpu.make_async_copy(row_hbm.at[pl.ds(0, CH)], ivv.at[b], sls[b]).wait()
        pltpu.make_async_copy(m_hbm.at[pl.ds(0, CH)], mvv.at[b], sls[b]).wait()

    def scat(b):
        pltpu.sync_copy(mvv.at[b], sh.at[ivv.at[b]], add=True)

    issue_load(0, 0)
    issue_load(1, 1)

    def pair(p_idx, carry):
        k0 = 2 * p_idx
        wait_load(0)
        scat(0)
        issue_load(k0 + 2, 0)
        wait_load(1)
        scat(1)

        @pl.when(k0 + 3 < NCH)
        def _():
            issue_load(k0 + 3, 1)

        return carry

    lax.fori_loop(0, (NCH - 1) // 2, pair, 0)
    wait_load(0)
    scat(0)
    plsc.subcore_barrier()
    pltpu.sync_copy(sh.at[pl.ds(ss * RPT, RPT)],
                    agg_hbm.at[cc, pl.ds(ss * RPT, RPT)])


def _node_body(h_ref, a0_ref, a1_ref, x_ref, dinv_ref,
               wh1a_ref, wh1b_ref, bh1_ref, wh2_ref, bh2_ref, g_ref, b_ref,
               hn_ref, xn_ref):
    hh = h_ref[...]
    acc = a0_ref[0] + a1_ref[0]
    agg = acc[:, :H]
    t = _silu(jnp.dot(hh, wh1a_ref[...], preferred_element_type=_f32)
              + jnp.dot(agg, wh1b_ref[...], preferred_element_type=_f32)
              + bh1_ref[...])
    hu = jnp.dot(t, wh2_ref[...], preferred_element_type=_f32) + bh2_ref[...]
    y = hh + hu
    mu = jnp.mean(y, axis=-1, keepdims=True)
    var = jnp.mean((y - mu) ** 2, axis=-1, keepdims=True)
    hn_ref[...] = (y - mu) * lax.rsqrt(var + 1e-5) * g_ref[...] + b_ref[...]
    xn_ref[...] = x_ref[...] + acc[:, H:H + 3] * dinv_ref[...]


def _node(h, a0, a1, x, dinv, wh1a, wh1b, b_h1, W_h2, b_h2, ln_g, ln_b):
    blk = 400
    full = lambda shape: pl.BlockSpec(shape, lambda i: (0, 0))
    return pl.pallas_call(
        _node_body,
        grid=(N // blk,),
        in_specs=[
            pl.BlockSpec((blk, D), lambda i: (i, 0)),
            pl.BlockSpec((1, blk, W), lambda i: (0, i, 0)),
            pl.BlockSpec((1, blk, W), lambda i: (1, i, 0)),
            pl.BlockSpec((blk, 3), lambda i: (i, 0)),
            pl.BlockSpec((blk, 1), lambda i: (i, 0)),
            full((D, H)), full((H, H)), full((1, H)),
            full((H, D)), full((1, D)), full((1, D)), full((1, D)),
        ],
        out_specs=[
            pl.BlockSpec((blk, D), lambda i: (i, 0)),
            pl.BlockSpec((blk, 3), lambda i: (i, 0)),
        ],
        out_shape=[
            jax.ShapeDtypeStruct((N, D), _f32),
            jax.ShapeDtypeStruct((N, 3), _f32),
        ],
    )(h, a0, a1, x, dinv, wh1a, wh1b, b_h1, W_h2, b_h2, ln_g, ln_b)


def kernel(h, x, edge_index, degree_inv,
           W_e1, b_e1, W_e2, b_e2, W_h1, b_h1, W_h2, b_h2,
           W_x1, b_x1, W_x2, b_x2, ln_g, ln_b):
    row = edge_index[0].astype(jnp.int32)
    col = edge_index[1].astype(jnp.int32)

    t1, t2 = _proj(h, x, W_e1[:D], W_e1[D:2 * D])
    p = _sc_gather(t1, t2, row, col)
    m = _edge_mlp(p, b_e1.reshape(1, H), W_e2, b_e2.reshape(1, H),
                  W_x1, b_x1.reshape(1, H), W_x2, b_x2.reshape(1, 1),
                  jnp.tile(W_e1[2 * D].reshape(1, H), (16, 1)))
    aggp = _sc_scatter(m, row)
    h_new, x_new = _node(h, aggp, aggp, x,
                         degree_inv.reshape(N, 1),
                         W_h1[:D], W_h1[D:], b_h1.reshape(1, H),
                         W_h2, b_h2.reshape(1, D), ln_g.reshape(1, D),
                         ln_b.reshape(1, D))
    return (h_new, x_new)


# speedup vs baseline: 8.8293x; 1.1175x over previous
import functools

import jax
import jax.numpy as jnp
from jax import lax
from jax.experimental import pallas as pl
from jax.experimental.pallas import tpu as pltpu
from jax.experimental.pallas import tpu_sc as plsc

N, E, D, H = 10000, 320000, 128, 64
NC, NS, L = 2, 16, 16
NW = NC * NS
W = 128
NP = 10240
RPT = NP // NS
EH = E // 2
EPT = EH // NW
CH = 40
NCH = EPT // CH

_mesh = plsc.VectorSubcoreMesh(core_axis_name="c", subcore_axis_name="s")
_f32 = jnp.float32


def _silu(v):
    return v / (1.0 + jnp.exp(-v))


def _proj_body(h_ref, x_ref, wa_ref, wb_ref, t1_ref, t2_ref):
    hh = h_ref[...]
    xb = x_ref[...]
    pad = jnp.zeros((hh.shape[0], W - H - 3), _f32)
    a = jnp.dot(hh, wa_ref[...], preferred_element_type=_f32)
    b = jnp.dot(hh, wb_ref[...], preferred_element_type=_f32)
    t1_ref[...] = jnp.concatenate([a, xb, pad], axis=1)
    t2_ref[...] = jnp.concatenate([b, xb, pad], axis=1)


def _proj(h, x, wa, wb):
    blk = 400
    return pl.pallas_call(
        _proj_body,
        grid=(N // blk,),
        in_specs=[
            pl.BlockSpec((blk, D), lambda i: (i, 0)),
            pl.BlockSpec((blk, 3), lambda i: (i, 0)),
            pl.BlockSpec((D, H), lambda i: (0, 0)),
            pl.BlockSpec((D, H), lambda i: (0, 0)),
        ],
        out_specs=[
            pl.BlockSpec((blk, W), lambda i: (i, 0)),
            pl.BlockSpec((blk, W), lambda i: (i, 0)),
        ],
        out_shape=[
            jax.ShapeDtypeStruct((N, W), _f32),
            jax.ShapeDtypeStruct((N, W), _f32),
        ],
    )(h, x, wa, wb)


def _make_sc_gather(offset):
    @functools.partial(
        pl.kernel,
        out_type=jax.ShapeDtypeStruct((EH, W), _f32),
        mesh=_mesh,
        scratch_types=[
            pltpu.VMEM((EPT,), jnp.int32),
            pltpu.VMEM((EPT,), jnp.int32),
            pltpu.VMEM((2, CH, W), _f32),
            pltpu.VMEM((2, CH, W), _f32),
            pltpu.VMEM((2, CH, W), _f32),
            pltpu.SemaphoreType.DMA,
            pltpu.SemaphoreType.DMA,
            pltpu.SemaphoreType.DMA,
            pltpu.SemaphoreType.DMA,
        ],
    )
    def sc_gather(t1_hbm, t2_hbm, row_hbm, col_hbm, p_hbm,
                  irf, icf, avv, bvv, pvv, sg0, sg1, sw0, sw1):
        wid = lax.axis_index("c") * NS + lax.axis_index("s")
        zero = jnp.zeros((L,), _f32)
        sgs = (sg0, sg1)
        sws = (sw0, sw1)

        def zrow(i, carry):
            for b in range(2):
                pvv[b, i, pl.ds(5 * L, L)] = zero
                pvv[b, i, pl.ds(6 * L, L)] = zero
                pvv[b, i, pl.ds(7 * L, L)] = zero
            return carry

        lax.fori_loop(0, CH, zrow, 0)
        pltpu.sync_copy(row_hbm.at[pl.ds(offset + wid * EPT, EPT)], irf)
        pltpu.sync_copy(col_hbm.at[pl.ds(offset + wid * EPT, EPT)], icf)

        def issue_gather(k, b):
            pltpu.async_copy(t1_hbm.at[irf.at[pl.ds(k * CH, CH)]], avv.at[b],
                             sgs[b])
            pltpu.async_copy(t2_hbm.at[icf.at[pl.ds(k * CH, CH)]], bvv.at[b],
                             sgs[b])

        def wait_gather(b):
            pltpu.make_async_copy(t1_hbm.at[pl.ds(0, CH)], avv.at[b], sgs[b]).wait()
            pltpu.make_async_copy(t2_hbm.at[pl.ds(0, CH)], bvv.at[b], sgs[b]).wait()

        def drain_wb(b):
            pltpu.make_async_copy(p_hbm.at[pl.ds(0, CH)], pvv.at[b], sws[b]).wait()

        def pack(b):
            av = avv.at[b]
            bv = bvv.at[b]
            pv = pvv.at[b]

            def pack_row(i, c2):
                for j in range(H // L):
                    sl = pl.ds(j * L, L)
                    pv[i, sl] = av[i, sl] + bv[i, sl]
                sl = pl.ds(H, L)
                pv[i, sl] = av[i, sl] - bv[i, sl]
                return c2

            lax.fori_loop(0, CH, pack_row, 0)

        def process(k, b, drain):
            wait_gather(b)

            @pl.when(drain)
            def _():
                drain_wb(b)

            pack(b)
            pltpu.async_copy(pvv.at[b], p_hbm.at[pl.ds(wid * EPT + k * CH, CH)],
                             sws[b])

        issue_gather(0, 0)
        issue_gather(1, 1)

        def pair(p_idx, carry):
            k0 = 2 * p_idx
            process(k0, 0, p_idx > 0)
            issue_gather(k0 + 2, 0)

            process(k0 + 1, 1, p_idx > 0)

            @pl.when(k0 + 3 < NCH)
            def _():
                issue_gather(k0 + 3, 1)

            return carry

        lax.fori_loop(0, (NCH - 1) // 2, pair, 0)
        process(NCH - 1, 0, True)
        drain_wb(0)
        drain_wb(1)

    return sc_gather


_sc_gather_a = _make_sc_gather(0)
_sc_gather_b = _make_sc_gather(EH)


def _edge_body(p_ref, be1_ref, we2_ref, be2_ref,
               wx1_ref, bx1_ref, wx2_ref, bx2_ref, wd2_ref, m_ref):
    p = p_ref[...]
    rel = p[:, H:H + 16]
    s = (p[:, :H] + be1_ref[...]
         + jnp.dot(rel * rel, wd2_ref[...], preferred_element_type=_f32))
    m = _silu(s)
    m = _silu(jnp.dot(m, we2_ref[...], preferred_element_type=_f32) + be2_ref[...])
    t = _silu(jnp.dot(m, wx1_ref[...], preferred_element_type=_f32) + bx1_ref[...])
    w_ij = jnp.dot(t, wx2_ref[...], preferred_element_type=_f32) + bx2_ref[...]
    pad = jnp.zeros((p.shape[0], W - H - 16), _f32)
    m_ref[...] = jnp.concatenate([m, rel * w_ij, pad], axis=1)


def _edge_mlp(p, b_e1, W_e2, b_e2, W_x1, b_x1, W_x2, b_x2, wd2_16):
    blk = 2000
    full = lambda shape: pl.BlockSpec(shape, lambda i: (0, 0))
    return pl.pallas_call(
        _edge_body,
        grid=(EH // blk,),
        in_specs=[
            pl.BlockSpec((blk, W), lambda i: (i, 0)),
            full((1, H)), full((H, H)), full((1, H)),
            full((H, H)), full((1, H)), full((H, 1)), full((1, 1)),
            full((16, H)),
        ],
        out_specs=pl.BlockSpec((blk, W), lambda i: (i, 0)),
        out_shape=jax.ShapeDtypeStruct((EH, W), _f32),
    )(p, b_e1, W_e2, b_e2, W_x1, b_x1, W_x2, b_x2, wd2_16)


def _make_sc_scatter(offset):
    @functools.partial(
        pl.kernel,
        out_type=jax.ShapeDtypeStruct((NC, NP, W), _f32),
        mesh=_mesh,
        scratch_types=[
            pltpu.VMEM_SHARED((NP, W), _f32),
            pltpu.VMEM((2, CH), jnp.int32),
            pltpu.VMEM((2, CH, W), _f32),
            pltpu.SemaphoreType.DMA,
            pltpu.SemaphoreType.DMA,
        ],
    )
    def sc_scatter(m_hbm, row_hbm, agg_hbm, sh, ivv, mvv, sl0, sl1):
        cc = lax.axis_index("c")
        ss = lax.axis_index("s")
        wid = cc * NS + ss
        sls = (sl0, sl1)
        zero = jnp.zeros((L,), _f32)

        def zrow(i, carry):
            for j in range(W // L):
                mvv[0, i, pl.ds(j * L, L)] = zero
            return carry

        lax.fori_loop(0, CH, zrow, 0)
        for r in range(RPT // CH):
            pltpu.sync_copy(mvv.at[0], sh.at[pl.ds(ss * RPT + r * CH, CH)])
        plsc.subcore_barrier()

        def issue_load(k, b):
            pltpu.async_copy(row_hbm.at[pl.ds(offset + wid * EPT + k * CH, CH)],
                             ivv.at[b], sls[b])
            pltpu.async_copy(m_hbm.at[pl.ds(wid * EPT + k * CH, CH)],
                             mvv.at[b], sls[b])

        def wait_load(b):
            pltpu.make_async_copy(row_hbm.at[pl.ds(0, CH)], ivv.at[b], sls[b]).wait()
            pltpu.make_async_copy(m_hbm.at[pl.ds(0, CH)], mvv.at[b], sls[b]).wait()

        def scat(b):
            pltpu.sync_copy(mvv.at[b], sh.at[ivv.at[b]], add=True)

        issue_load(0, 0)
        issue_load(1, 1)

        def pair(p_idx, carry):
            k0 = 2 * p_idx
            wait_load(0)
            scat(0)
            issue_load(k0 + 2, 0)
            wait_load(1)
            scat(1)

            @pl.when(k0 + 3 < NCH)
            def _():
                issue_load(k0 + 3, 1)

            return carry

        lax.fori_loop(0, (NCH - 1) // 2, pair, 0)
        wait_load(0)
        scat(0)
        plsc.subcore_barrier()
        pltpu.sync_copy(sh.at[pl.ds(ss * RPT, RPT)],
                        agg_hbm.at[cc, pl.ds(ss * RPT, RPT)])

    return sc_scatter


_sc_scatter_a = _make_sc_scatter(0)
_sc_scatter_b = _make_sc_scatter(EH)


def _node_body(h_ref, a0_ref, a1_ref, a2_ref, a3_ref, x_ref, dinv_ref,
               wh1a_ref, wh1b_ref, bh1_ref, wh2_ref, bh2_ref, g_ref, b_ref,
               hn_ref, xn_ref):
    hh = h_ref[...]
    acc = a0_ref[0] + a1_ref[0] + a2_ref[0] + a3_ref[0]
    agg = acc[:, :H]
    t = _silu(jnp.dot(hh, wh1a_ref[...], preferred_element_type=_f32)
              + jnp.dot(agg, wh1b_ref[...], preferred_element_type=_f32)
              + bh1_ref[...])
    hu = jnp.dot(t, wh2_ref[...], preferred_element_type=_f32) + bh2_ref[...]
    y = hh + hu
    mu = jnp.mean(y, axis=-1, keepdims=True)
    var = jnp.mean((y - mu) ** 2, axis=-1, keepdims=True)
    hn_ref[...] = (y - mu) * lax.rsqrt(var + 1e-5) * g_ref[...] + b_ref[...]
    xn_ref[...] = x_ref[...] + acc[:, H:H + 3] * dinv_ref[...]


def _node(h, agg1, agg2, x, dinv, wh1a, wh1b, b_h1, W_h2, b_h2, ln_g, ln_b):
    blk = 400
    full = lambda shape: pl.BlockSpec(shape, lambda i: (0, 0))
    part0 = pl.BlockSpec((1, blk, W), lambda i: (0, i, 0))
    part1 = pl.BlockSpec((1, blk, W), lambda i: (1, i, 0))
    return pl.pallas_call(
        _node_body,
        grid=(N // blk,),
        in_specs=[
            pl.BlockSpec((blk, D), lambda i: (i, 0)),
            part0, part1, part0, part1,
            pl.BlockSpec((blk, 3), lambda i: (i, 0)),
            pl.BlockSpec((blk, 1), lambda i: (i, 0)),
            full((D, H)), full((H, H)), full((1, H)),
            full((H, D)), full((1, D)), full((1, D)), full((1, D)),
        ],
        out_specs=[
            pl.BlockSpec((blk, D), lambda i: (i, 0)),
            pl.BlockSpec((blk, 3), lambda i: (i, 0)),
        ],
        out_shape=[
            jax.ShapeDtypeStruct((N, D), _f32),
            jax.ShapeDtypeStruct((N, 3), _f32),
        ],
    )(h, agg1, agg1, agg2, agg2, x, dinv,
      wh1a, wh1b, b_h1, W_h2, b_h2, ln_g, ln_b)


def kernel(h, x, edge_index, degree_inv,
           W_e1, b_e1, W_e2, b_e2, W_h1, b_h1, W_h2, b_h2,
           W_x1, b_x1, W_x2, b_x2, ln_g, ln_b):
    row = edge_index[0].astype(jnp.int32)
    col = edge_index[1].astype(jnp.int32)

    t1, t2 = _proj(h, x, W_e1[:D], W_e1[D:2 * D])
    ew = (b_e1.reshape(1, H), W_e2, b_e2.reshape(1, H),
          W_x1, b_x1.reshape(1, H), W_x2, b_x2.reshape(1, 1),
          jnp.tile(W_e1[2 * D].reshape(1, H), (16, 1)))

    p1 = _sc_gather_a(t1, t2, row, col)
    m1 = _edge_mlp(p1, *ew)
    p2 = _sc_gather_b(t1, t2, row, col)
    agg1 = _sc_scatter_a(m1, row)
    m2 = _edge_mlp(p2, *ew)
    agg2 = _sc_scatter_b(m2, row)

    h_new, x_new = _node(h, agg1, agg2, x,
                         degree_inv.reshape(N, 1),
                         W_h1[:D], W_h1[D:], b_h1.reshape(1, H),
                         W_h2, b_h2.reshape(1, D), ln_g.reshape(1, D),
                         ln_b.reshape(1, D))
    return (h_new, x_new)


# speedup vs baseline: 9.5604x; 1.0828x over previous
import functools

import jax
import jax.numpy as jnp
from jax import lax
from jax.experimental import pallas as pl
from jax.experimental.pallas import tpu as pltpu
from jax.experimental.pallas import tpu_sc as plsc

N, E, D, H = 10000, 320000, 128, 64
NC, NS, L = 2, 16, 16
NW = NC * NS
W = 128
NP = 10240
RPT = NP // NS
EH = E // 2
EPT = EH // NW
CH = 40
NCH = EPT // CH
NB = 4

_mesh = plsc.VectorSubcoreMesh(core_axis_name="c", subcore_axis_name="s")
_f32 = jnp.float32


def _silu(v):
    return v / (1.0 + jnp.exp(-v))


def _proj_body(h_ref, x_ref, wa_ref, wb_ref, t1_ref, t2_ref):
    hh = h_ref[...]
    xb = x_ref[...]
    pad = jnp.zeros((hh.shape[0], W - H - 3), _f32)
    a = jnp.dot(hh, wa_ref[...], preferred_element_type=_f32)
    b = jnp.dot(hh, wb_ref[...], preferred_element_type=_f32)
    t1_ref[...] = jnp.concatenate([a, xb, pad], axis=1)
    t2_ref[...] = jnp.concatenate([b, xb, pad], axis=1)


def _proj(h, x, wa, wb):
    blk = 400
    return pl.pallas_call(
        _proj_body,
        grid=(N // blk,),
        in_specs=[
            pl.BlockSpec((blk, D), lambda i: (i, 0)),
            pl.BlockSpec((blk, 3), lambda i: (i, 0)),
            pl.BlockSpec((D, H), lambda i: (0, 0)),
            pl.BlockSpec((D, H), lambda i: (0, 0)),
        ],
        out_specs=[
            pl.BlockSpec((blk, W), lambda i: (i, 0)),
            pl.BlockSpec((blk, W), lambda i: (i, 0)),
        ],
        out_shape=[
            jax.ShapeDtypeStruct((N, W), _f32),
            jax.ShapeDtypeStruct((N, W), _f32),
        ],
    )(h, x, wa, wb)


def _make_sc_gather(offset):
    @functools.partial(
        pl.kernel,
        out_type=jax.ShapeDtypeStruct((EH, W), _f32),
        mesh=_mesh,
        scratch_types=[
            pltpu.VMEM((EPT,), jnp.int32),
            pltpu.VMEM((EPT,), jnp.int32),
            pltpu.VMEM((NB, CH, W), _f32),
            pltpu.VMEM((NB, CH, W), _f32),
            pltpu.VMEM((NB, CH, W), _f32),
        ] + [pltpu.SemaphoreType.DMA] * (2 * NB),
    )
    def sc_gather(t1_hbm, t2_hbm, row_hbm, col_hbm, p_hbm,
                  irf, icf, avv, bvv, pvv, *sems):
        wid = lax.axis_index("c") * NS + lax.axis_index("s")
        zero = jnp.zeros((L,), _f32)
        sgs = sems[:NB]
        sws = sems[NB:]

        def zrow(i, carry):
            for b in range(NB):
                pvv[b, i, pl.ds(5 * L, L)] = zero
                pvv[b, i, pl.ds(6 * L, L)] = zero
                pvv[b, i, pl.ds(7 * L, L)] = zero
            return carry

        lax.fori_loop(0, CH, zrow, 0)
        pltpu.sync_copy(row_hbm.at[pl.ds(offset + wid * EPT, EPT)], irf)
        pltpu.sync_copy(col_hbm.at[pl.ds(offset + wid * EPT, EPT)], icf)

        def issue_gather(k, b):
            pltpu.async_copy(t1_hbm.at[irf.at[pl.ds(k * CH, CH)]], avv.at[b],
                             sgs[b])
            pltpu.async_copy(t2_hbm.at[icf.at[pl.ds(k * CH, CH)]], bvv.at[b],
                             sgs[b])

        def wait_gather(b):
            pltpu.make_async_copy(t1_hbm.at[pl.ds(0, CH)], avv.at[b], sgs[b]).wait()
            pltpu.make_async_copy(t2_hbm.at[pl.ds(0, CH)], bvv.at[b], sgs[b]).wait()

        def drain_wb(b):
            pltpu.make_async_copy(p_hbm.at[pl.ds(0, CH)], pvv.at[b], sws[b]).wait()

        def pack(b):
            av = avv.at[b]
            bv = bvv.at[b]
            pv = pvv.at[b]

            def pack_row(i, c2):
                for j in range(H // L):
                    sl = pl.ds(j * L, L)
                    pv[i, sl] = av[i, sl] + bv[i, sl]
                sl = pl.ds(H, L)
                pv[i, sl] = av[i, sl] - bv[i, sl]
                return c2

            lax.fori_loop(0, CH, pack_row, 0)

        def process(k, b, drain):
            wait_gather(b)

            @pl.when(drain)
            def _():
                drain_wb(b)

            pack(b)
            pltpu.async_copy(pvv.at[b], p_hbm.at[pl.ds(wid * EPT + k * CH, CH)],
                             sws[b])

        for b in range(NB):
            issue_gather(b, b)

        def quad(p_idx, carry):
            k0 = NB * p_idx
            for b in range(NB):
                process(k0 + b, b, p_idx > 0)

                @pl.when(k0 + b + NB < NCH)
                def _():
                    issue_gather(k0 + b + NB, b)

            return carry

        lax.fori_loop(0, NCH // NB, quad, 0)
        for b in range(NCH % NB):
            process((NCH // NB) * NB + b, b, True)
        for b in range(NB):
            drain_wb(b)

    return sc_gather


_sc_gather_a = _make_sc_gather(0)
_sc_gather_b = _make_sc_gather(EH)


def _edge_body(p_ref, be1_ref, we2_ref, be2_ref,
               wx1_ref, bx1_ref, wx2_ref, bx2_ref, wd2_ref, m_ref):
    p = p_ref[...]
    rel = p[:, H:H + 16]
    s = (p[:, :H] + be1_ref[...]
         + jnp.dot(rel * rel, wd2_ref[...], preferred_element_type=_f32))
    m = _silu(s)
    m = _silu(jnp.dot(m, we2_ref[...], preferred_element_type=_f32) + be2_ref[...])
    t = _silu(jnp.dot(m, wx1_ref[...], preferred_element_type=_f32) + bx1_ref[...])
    w_ij = jnp.dot(t, wx2_ref[...], preferred_element_type=_f32) + bx2_ref[...]
    pad = jnp.zeros((p.shape[0], W - H - 16), _f32)
    m_ref[...] = jnp.concatenate([m, rel * w_ij, pad], axis=1)


def _edge_mlp(p, b_e1, W_e2, b_e2, W_x1, b_x1, W_x2, b_x2, wd2_16):
    blk = 2000
    full = lambda shape: pl.BlockSpec(shape, lambda i: (0, 0))
    return pl.pallas_call(
        _edge_body,
        grid=(EH // blk,),
        in_specs=[
            pl.BlockSpec((blk, W), lambda i: (i, 0)),
            full((1, H)), full((H, H)), full((1, H)),
            full((H, H)), full((1, H)), full((H, 1)), full((1, 1)),
            full((16, H)),
        ],
        out_specs=pl.BlockSpec((blk, W), lambda i: (i, 0)),
        out_shape=jax.ShapeDtypeStruct((EH, W), _f32),
    )(p, b_e1, W_e2, b_e2, W_x1, b_x1, W_x2, b_x2, wd2_16)


def _make_sc_scatter(offset):
    @functools.partial(
        pl.kernel,
        out_type=jax.ShapeDtypeStruct((NC, NP, W), _f32),
        mesh=_mesh,
        scratch_types=[
            pltpu.VMEM_SHARED((NP, W), _f32),
            pltpu.VMEM((NB, CH), jnp.int32),
            pltpu.VMEM((NB, CH, W), _f32),
        ] + [pltpu.SemaphoreType.DMA] * NB,
    )
    def sc_scatter(m_hbm, row_hbm, agg_hbm, sh, ivv, mvv, *sls):
        cc = lax.axis_index("c")
        ss = lax.axis_index("s")
        wid = cc * NS + ss
        zero = jnp.zeros((L,), _f32)

        def zrow(i, carry):
            for j in range(W // L):
                mvv[0, i, pl.ds(j * L, L)] = zero
            return carry

        lax.fori_loop(0, CH, zrow, 0)
        zoff = 0
        while zoff < RPT:
            zn = min(CH, RPT - zoff)
            pltpu.sync_copy(mvv.at[0, pl.ds(0, zn)],
                            sh.at[pl.ds(ss * RPT + zoff, zn)])
            zoff += zn
        plsc.subcore_barrier()

        def issue_load(k, b):
            pltpu.async_copy(row_hbm.at[pl.ds(offset + wid * EPT + k * CH, CH)],
                             ivv.at[b], sls[b])
            pltpu.async_copy(m_hbm.at[pl.ds(wid * EPT + k * CH, CH)],
                             mvv.at[b], sls[b])

        def wait_load(b):
            pltpu.make_async_copy(row_hbm.at[pl.ds(0, CH)], ivv.at[b], sls[b]).wait()
            pltpu.make_async_copy(m_hbm.at[pl.ds(0, CH)], mvv.at[b], sls[b]).wait()

        def scat(b):
            pltpu.sync_copy(mvv.at[b], sh.at[ivv.at[b]], add=True)

        for b in range(NB):
            issue_load(b, b)

        def quad(p_idx, carry):
            k0 = NB * p_idx
            for b in range(NB):
                wait_load(b)
                scat(b)

                @pl.when(k0 + b + NB < NCH)
                def _():
                    issue_load(k0 + b + NB, b)

            return carry

        lax.fori_loop(0, NCH // NB, quad, 0)
        for b in range(NCH % NB):
            wait_load(b)
            scat(b)
        plsc.subcore_barrier()
        pltpu.sync_copy(sh.at[pl.ds(ss * RPT, RPT)],
                        agg_hbm.at[cc, pl.ds(ss * RPT, RPT)])

    return sc_scatter


_sc_scatter_a = _make_sc_scatter(0)
_sc_scatter_b = _make_sc_scatter(EH)


def _node_body(h_ref, a0_ref, a1_ref, a2_ref, a3_ref, x_ref, dinv_ref,
               wh1a_ref, wh1b_ref, bh1_ref, wh2_ref, bh2_ref, g_ref, b_ref,
               hn_ref, xn_ref):
    hh = h_ref[...]
    acc = a0_ref[0] + a1_ref[0] + a2_ref[0] + a3_ref[0]
    agg = acc[:, :H]
    t = _silu(jnp.dot(hh, wh1a_ref[...], preferred_element_type=_f32)
              + jnp.dot(agg, wh1b_ref[...], preferred_element_type=_f32)
              + bh1_ref[...])
    hu = jnp.dot(t, wh2_ref[...], preferred_element_type=_f32) + bh2_ref[...]
    y = hh + hu
    mu = jnp.mean(y, axis=-1, keepdims=True)
    var = jnp.mean((y - mu) ** 2, axis=-1, keepdims=True)
    hn_ref[...] = (y - mu) * lax.rsqrt(var + 1e-5) * g_ref[...] + b_ref[...]
    xn_ref[...] = x_ref[...] + acc[:, H:H + 3] * dinv_ref[...]


def _node(h, agg1, agg2, x, dinv, wh1a, wh1b, b_h1, W_h2, b_h2, ln_g, ln_b):
    blk = 400
    full = lambda shape: pl.BlockSpec(shape, lambda i: (0, 0))
    part0 = pl.BlockSpec((1, blk, W), lambda i: (0, i, 0))
    part1 = pl.BlockSpec((1, blk, W), lambda i: (1, i, 0))
    return pl.pallas_call(
        _node_body,
        grid=(N // blk,),
        in_specs=[
            pl.BlockSpec((blk, D), lambda i: (i, 0)),
            part0, part1, part0, part1,
            pl.BlockSpec((blk, 3), lambda i: (i, 0)),
            pl.BlockSpec((blk, 1), lambda i: (i, 0)),
            full((D, H)), full((H, H)), full((1, H)),
            full((H, D)), full((1, D)), full((1, D)), full((1, D)),
        ],
        out_specs=[
            pl.BlockSpec((blk, D), lambda i: (i, 0)),
            pl.BlockSpec((blk, 3), lambda i: (i, 0)),
        ],
        out_shape=[
            jax.ShapeDtypeStruct((N, D), _f32),
            jax.ShapeDtypeStruct((N, 3), _f32),
        ],
    )(h, agg1, agg1, agg2, agg2, x, dinv,
      wh1a, wh1b, b_h1, W_h2, b_h2, ln_g, ln_b)


def kernel(h, x, edge_index, degree_inv,
           W_e1, b_e1, W_e2, b_e2, W_h1, b_h1, W_h2, b_h2,
           W_x1, b_x1, W_x2, b_x2, ln_g, ln_b):
    row = edge_index[0].astype(jnp.int32)
    col = edge_index[1].astype(jnp.int32)

    t1, t2 = _proj(h, x, W_e1[:D], W_e1[D:2 * D])
    ew = (b_e1.reshape(1, H), W_e2, b_e2.reshape(1, H),
          W_x1, b_x1.reshape(1, H), W_x2, b_x2.reshape(1, 1),
          jnp.tile(W_e1[2 * D].reshape(1, H), (16, 1)))

    p1 = _sc_gather_a(t1, t2, row, col)
    m1 = _edge_mlp(p1, *ew)
    p2 = _sc_gather_b(t1, t2, row, col)
    agg1 = _sc_scatter_a(m1, row)
    m2 = _edge_mlp(p2, *ew)
    agg2 = _sc_scatter_b(m2, row)

    h_new, x_new = _node(h, agg1, agg2, x,
                         degree_inv.reshape(N, 1),
                         W_h1[:D], W_h1[D:], b_h1.reshape(1, H),
                         W_h2, b_h2.reshape(1, D), ln_g.reshape(1, D),
                         ln_b.reshape(1, D))
    return (h_new, x_new)
